# Initial kernel scaffold; baseline (speedup 1.0000x reference)
import jax
import jax.numpy as jnp
from jax.experimental import pallas as pl


def kernel(x, table, Wn, bn, W1, b1, g1, be1, W2, b2, g2, be2):
    raise NotImplementedError("write your pallas kernel here")


# speedup vs baseline: 10.0931x; 10.0931x over previous
import functools

import jax
import jax.numpy as jnp
import numpy as np
from jax.experimental import pallas as pl

FIELD_DIMS = (9, 4, 7, 2, 20, 7, 50, 8, 8, 2, 2, 2, 2, 2, 2, 2)
_OFFS = tuple(int(v) for v in np.cumsum((0,) + FIELD_DIMS[:-1]))
NFIELD = 16
EMBED = 128
H1, H2 = 512, 256
EPS = 1e-5
BLK = 1024


def _prep_kernel(table_ref, wc_ref, w1n_ref, wnt_ref, bn_ref, b1_ref,
                 pall_ref, m_ref, b1p_ref):
    rows = jax.lax.broadcasted_iota(jnp.int32, (136, H1), 0)
    acc = jnp.zeros((136, H1), jnp.float32)
    for f in range(NFIELD):
        pf = jnp.dot(table_ref[...], wc_ref[f * EMBED:(f + 1) * EMBED, :],
                     preferred_element_type=jnp.float32)
        mask = (rows >= _OFFS[f]) & (rows < _OFFS[f] + FIELD_DIMS[f])
        acc = acc + jnp.where(mask, pf, 0.0)
    pall_ref[...] = jnp.concatenate(
        [acc, jnp.zeros((120, H1), jnp.float32)], axis=0)
    m_ref[...] = jnp.dot(wnt_ref[...], w1n_ref[...],
                         preferred_element_type=jnp.float32)
    b1p = b1_ref[...] + jnp.dot(bn_ref[...], w1n_ref[...],
                                preferred_element_type=jnp.float32)
    b1p_ref[...] = jnp.broadcast_to(b1p, (8, H1))


def _pass1_kernel(x_ref, pall_ref, m_ref, b1p_ref, h1_ref, st_ref):
    i = pl.program_id(0)
    blk = x_ref.shape[0]
    xb = x_ref[...]
    lane = jax.lax.broadcasted_iota(jnp.int32, (blk, 256), 1).astype(jnp.float32)
    u = jnp.zeros((blk, 256), jnp.float32)
    for f in range(NFIELD):
        u = u + (lane == (xb[:, f:f + 1] + float(_OFFS[f]))).astype(jnp.float32)
    xnum = jnp.concatenate(
        [xb[:, NFIELD:], jnp.zeros((blk, 1), jnp.float32)], axis=1)
    h = (jnp.dot(u, pall_ref[...], preferred_element_type=jnp.float32)
         + jnp.dot(xnum, m_ref[...], preferred_element_type=jnp.float32)
         + b1p_ref[0:1, :])
    h1_ref[...] = h
    s = jnp.sum(h, axis=0, keepdims=True)
    ss = jnp.sum(h * h, axis=0, keepdims=True)
    stat = jnp.concatenate([s, ss, jnp.zeros((6, H1), jnp.float32)], axis=0)

    @pl.when(i == 0)
    def _():
        st_ref[...] = stat

    @pl.when(i > 0)
    def _():
        st_ref[...] += stat


def _pass2_kernel(h1_ref, st_ref, g1_ref, be1_ref, w2t_ref, b2_ref,
                  h2_ref, st2_ref, *, inv_b):
    i = pl.program_id(0)
    mu = st_ref[0:1, :] * inv_b
    var = st_ref[1:2, :] * inv_b - mu * mu
    a = g1_ref[...] * jax.lax.rsqrt(var + EPS)
    c = be1_ref[...] - mu * a
    h1 = jnp.maximum(h1_ref[...] * a + c, 0.0)
    h2 = jnp.dot(h1, w2t_ref[...], preferred_element_type=jnp.float32) \
        + b2_ref[...]
    h2_ref[...] = h2
    s = jnp.sum(h2, axis=0, keepdims=True)
    ss = jnp.sum(h2 * h2, axis=0, keepdims=True)
    stat = jnp.concatenate([s, ss, jnp.zeros((6, H2), jnp.float32)], axis=0)

    @pl.when(i == 0)
    def _():
        st2_ref[...] = stat

    @pl.when(i > 0)
    def _():
        st2_ref[...] += stat


def _pass3_kernel(h2_ref, st2_ref, g2_ref, be2_ref, out_ref, *, inv_b):
    mu = st2_ref[0:1, :] * inv_b
    var = st2_ref[1:2, :] * inv_b - mu * mu
    a = g2_ref[...] * jax.lax.rsqrt(var + EPS)
    c = be2_ref[...] - mu * a
    out_ref[...] = jnp.maximum(h2_ref[...] * a + c, 0.0)


def kernel(x, table, Wn, bn, W1, b1, g1, be1, W2, b2, g2, be2):
    bsz, nx = x.shape
    vocab = table.shape[0]
    table_pad = jnp.pad(table, ((0, 136 - vocab), (0, 0)))
    wc = jnp.transpose(
        W1[:, :NFIELD * EMBED].reshape(H1, NFIELD, EMBED),
        (1, 2, 0)).reshape(NFIELD * EMBED, H1)
    w1n = W1[:, NFIELD * EMBED:].T
    wnt = jnp.pad(Wn.T, ((0, 1), (0, 0)))

    pall, m, b1p = pl.pallas_call(
        _prep_kernel,
        out_shape=[jax.ShapeDtypeStruct((256, H1), jnp.float32),
                   jax.ShapeDtypeStruct((64, H1), jnp.float32),
                   jax.ShapeDtypeStruct((8, H1), jnp.float32)],
    )(table_pad, wc, w1n, wnt, bn.reshape(1, EMBED), b1.reshape(1, H1))

    nb = bsz // BLK
    h1pre, st1 = pl.pallas_call(
        _pass1_kernel,
        grid=(nb,),
        in_specs=[pl.BlockSpec((BLK, nx), lambda i: (i, 0)),
                  pl.BlockSpec((256, H1), lambda i: (0, 0)),
                  pl.BlockSpec((64, H1), lambda i: (0, 0)),
                  pl.BlockSpec((8, H1), lambda i: (0, 0))],
        out_specs=[pl.BlockSpec((BLK, H1), lambda i: (i, 0)),
                   pl.BlockSpec((8, H1), lambda i: (0, 0))],
        out_shape=[jax.ShapeDtypeStruct((bsz, H1), jnp.float32),
                   jax.ShapeDtypeStruct((8, H1), jnp.float32)],
    )(x, pall, m, b1p)

    h2pre, st2 = pl.pallas_call(
        functools.partial(_pass2_kernel, inv_b=1.0 / bsz),
        grid=(nb,),
        in_specs=[pl.BlockSpec((BLK, H1), lambda i: (i, 0)),
                  pl.BlockSpec((8, H1), lambda i: (0, 0)),
                  pl.BlockSpec((1, H1), lambda i: (0, 0)),
                  pl.BlockSpec((1, H1), lambda i: (0, 0)),
                  pl.BlockSpec((H1, H2), lambda i: (0, 0)),
                  pl.BlockSpec((1, H2), lambda i: (0, 0))],
        out_specs=[pl.BlockSpec((BLK, H2), lambda i: (i, 0)),
                   pl.BlockSpec((8, H2), lambda i: (0, 0))],
        out_shape=[jax.ShapeDtypeStruct((bsz, H2), jnp.float32),
                   jax.ShapeDtypeStruct((8, H2), jnp.float32)],
    )(h1pre, st1, g1.reshape(1, H1), be1.reshape(1, H1), W2.T,
      b2.reshape(1, H2))

    out = pl.pallas_call(
        functools.partial(_pass3_kernel, inv_b=1.0 / bsz),
        grid=(nb,),
        in_specs=[pl.BlockSpec((BLK, H2), lambda i: (i, 0)),
                  pl.BlockSpec((8, H2), lambda i: (0, 0)),
                  pl.BlockSpec((1, H2), lambda i: (0, 0)),
                  pl.BlockSpec((1, H2), lambda i: (0, 0))],
        out_specs=pl.BlockSpec((BLK, H2), lambda i: (i, 0)),
        out_shape=jax.ShapeDtypeStruct((bsz, H2), jnp.float32),
    )(h2pre, st2, g2.reshape(1, H2), be2.reshape(1, H2))
    return out
